# initial kernel scaffold (unmeasured)
import jax
import jax.numpy as jnp
from jax import lax
from jax.experimental import pallas as pl
from jax.experimental.pallas import tpu as pltpu


def kernel(Q, K, V):
    b, s, h, d = Q.shape
    scale = d ** -0.5

    def comm_body(k_ref, v_ref, kr_ref, vr_ref, send_sems, recv_sems):
        my_x = lax.axis_index("x")
        my_y = lax.axis_index("y")
        my_z = lax.axis_index("z")
        peer = (my_x, my_y, 1 - my_z)

        barrier = pltpu.get_barrier_semaphore()
        pl.semaphore_signal(
            barrier, inc=1, device_id=peer,
            device_id_type=pl.DeviceIdType.MESH,
        )
        pl.semaphore_wait(barrier, 1)

        rdma_k = pltpu.make_async_remote_copy(
            src_ref=k_ref,
            dst_ref=kr_ref,
            send_sem=send_sems.at[0],
            recv_sem=recv_sems.at[0],
            device_id=peer,
            device_id_type=pl.DeviceIdType.MESH,
        )
        rdma_v = pltpu.make_async_remote_copy(
            src_ref=v_ref,
            dst_ref=vr_ref,
            send_sem=send_sems.at[1],
            recv_sem=recv_sems.at[1],
            device_id=peer,
            device_id_type=pl.DeviceIdType.MESH,
        )
        rdma_k.start()
        rdma_v.start()
        rdma_k.wait()
        rdma_v.wait()

    Kr, Vr = pl.pallas_call(
        comm_body,
        out_shape=(
            jax.ShapeDtypeStruct(K.shape, K.dtype),
            jax.ShapeDtypeStruct(V.shape, V.dtype),
        ),
        in_specs=[
            pl.BlockSpec(memory_space=pltpu.MemorySpace.ANY),
            pl.BlockSpec(memory_space=pltpu.MemorySpace.ANY),
        ],
        out_specs=(
            pl.BlockSpec(memory_space=pltpu.MemorySpace.ANY),
            pl.BlockSpec(memory_space=pltpu.MemorySpace.ANY),
        ),
        scratch_shapes=[
            pltpu.SemaphoreType.DMA((2,)),
            pltpu.SemaphoreType.DMA((2,)),
        ],
        compiler_params=pltpu.CompilerParams(
            collective_id=0, has_side_effects=True
        ),
    )(K, V)

    def attn_body(q_ref, k1_ref, v1_ref, k2_ref, v2_ref, o_ref):
        q = q_ref[0, :, 0, :]
        k1 = k1_ref[0, :, 0, :]
        v1 = v1_ref[0, :, 0, :]
        k2 = k2_ref[0, :, 0, :]
        v2 = v2_ref[0, :, 0, :]

        dims = (((1,), (1,)), ((), ()))
        s1 = lax.dot_general(q, k1, dims, preferred_element_type=jnp.float32)
        s2 = lax.dot_general(q, k2, dims, preferred_element_type=jnp.float32)
        s1 = s1 * scale
        s2 = s2 * scale
        m = jnp.maximum(
            jnp.max(s1, axis=-1, keepdims=True),
            jnp.max(s2, axis=-1, keepdims=True),
        )
        p1 = jnp.exp(s1 - m)
        p2 = jnp.exp(s2 - m)
        l = (
            jnp.sum(p1, axis=-1, keepdims=True)
            + jnp.sum(p2, axis=-1, keepdims=True)
        )
        o = (
            lax.dot_general(p1, v1, (((1,), (0,)), ((), ())),
                            preferred_element_type=jnp.float32)
            + lax.dot_general(p2, v2, (((1,), (0,)), ((), ())),
                              preferred_element_type=jnp.float32)
        ) / l
        o_ref[0, :, 0, :] = o.astype(o_ref.dtype)

    hspec = pl.BlockSpec((1, s, 1, d), lambda i: (0, 0, i, 0))
    out = pl.pallas_call(
        attn_body,
        grid=(h,),
        in_specs=[hspec] * 5,
        out_specs=hspec,
        out_shape=jax.ShapeDtypeStruct(Q.shape, jnp.float32),
    )(Q, K, V, Kr, Vr)
    return out


# baseline (device time: 315152 ns/iter reference)
import jax
import jax.numpy as jnp
from jax import lax
from jax.experimental import pallas as pl
from jax.experimental.pallas import tpu as pltpu


def kernel(Q, K, V):
    b, s, h, d = Q.shape
    scale = d ** -0.5

    def comm_body(k_ref, v_ref, kr_ref, vr_ref, send_sems, recv_sems):
        my_x = lax.axis_index("x")
        my_y = lax.axis_index("y")
        my_z = lax.axis_index("z")
        peer = (my_x, my_y, 1 - my_z)

        barrier = pltpu.get_barrier_semaphore()
        pl.semaphore_signal(
            barrier, inc=1, device_id=peer,
            device_id_type=pl.DeviceIdType.MESH,
        )
        pl.semaphore_wait(barrier, 1)

        rdma_k = pltpu.make_async_remote_copy(
            src_ref=k_ref,
            dst_ref=kr_ref,
            send_sem=send_sems.at[0],
            recv_sem=recv_sems.at[0],
            device_id=peer,
            device_id_type=pl.DeviceIdType.MESH,
        )
        rdma_v = pltpu.make_async_remote_copy(
            src_ref=v_ref,
            dst_ref=vr_ref,
            send_sem=send_sems.at[1],
            recv_sem=recv_sems.at[1],
            device_id=peer,
            device_id_type=pl.DeviceIdType.MESH,
        )
        rdma_k.start()
        rdma_v.start()
        rdma_k.wait()
        rdma_v.wait()

    Kr, Vr = pl.pallas_call(
        comm_body,
        out_shape=(
            jax.ShapeDtypeStruct(K.shape, K.dtype),
            jax.ShapeDtypeStruct(V.shape, V.dtype),
        ),
        in_specs=[
            pl.BlockSpec(memory_space=pl.ANY),
            pl.BlockSpec(memory_space=pl.ANY),
        ],
        out_specs=(
            pl.BlockSpec(memory_space=pl.ANY),
            pl.BlockSpec(memory_space=pl.ANY),
        ),
        scratch_shapes=[
            pltpu.SemaphoreType.DMA((2,)),
            pltpu.SemaphoreType.DMA((2,)),
        ],
        compiler_params=pltpu.CompilerParams(
            collective_id=0, has_side_effects=True
        ),
    )(K, V)

    def attn_body(q_ref, k1_ref, v1_ref, k2_ref, v2_ref, o_ref):
        q = q_ref[:, :]
        k1 = k1_ref[:, :]
        v1 = v1_ref[:, :]
        k2 = k2_ref[:, :]
        v2 = v2_ref[:, :]

        dims = (((1,), (1,)), ((), ()))
        s1 = lax.dot_general(q, k1, dims, preferred_element_type=jnp.float32)
        s2 = lax.dot_general(q, k2, dims, preferred_element_type=jnp.float32)
        s1 = s1 * scale
        s2 = s2 * scale
        m = jnp.maximum(
            jnp.max(s1, axis=-1, keepdims=True),
            jnp.max(s2, axis=-1, keepdims=True),
        )
        p1 = jnp.exp(s1 - m)
        p2 = jnp.exp(s2 - m)
        l = (
            jnp.sum(p1, axis=-1, keepdims=True)
            + jnp.sum(p2, axis=-1, keepdims=True)
        )
        o = (
            lax.dot_general(p1, v1, (((1,), (0,)), ((), ())),
                            preferred_element_type=jnp.float32)
            + lax.dot_general(p2, v2, (((1,), (0,)), ((), ())),
                              preferred_element_type=jnp.float32)
        ) / l
        o_ref[:, :] = o.astype(o_ref.dtype)

    hspec = pl.BlockSpec((s, d), lambda i: (0, i))
    out = pl.pallas_call(
        attn_body,
        grid=(h,),
        in_specs=[hspec] * 5,
        out_specs=hspec,
        out_shape=jax.ShapeDtypeStruct((s, h * d), jnp.float32),
    )(
        Q.reshape(s, h * d),
        K.reshape(s, h * d),
        V.reshape(s, h * d),
        Kr.reshape(s, h * d),
        Vr.reshape(s, h * d),
    )
    return out.reshape(b, s, h, d)


# device time: 103345 ns/iter; 3.0495x vs baseline; 3.0495x over previous
import jax
import jax.numpy as jnp
from jax import lax
from jax.experimental import pallas as pl
from jax.experimental.pallas import tpu as pltpu


def kernel(Q, K, V):
    b, s, h, d = Q.shape
    hd = h * d
    half = h // 2
    scale = d ** -0.5
    f32 = jnp.float32

    NT = (((1,), (1,)), ((), ()))
    NN = (((1,), (0,)), ((), ()))

    def body(q_ref, k_ref, v_ref, o_ref, kr, vr, zs, zr, xs, xr):
        my_x = lax.axis_index("x")
        my_y = lax.axis_index("y")
        my_z = lax.axis_index("z")
        zpeer = (my_x, my_y, 1 - my_z)
        xpeer = (1 - my_x, my_y, my_z)

        barrier = pltpu.get_barrier_semaphore()
        pl.semaphore_signal(
            barrier, inc=1, device_id=zpeer,
            device_id_type=pl.DeviceIdType.MESH,
        )
        pl.semaphore_signal(
            barrier, inc=1, device_id=xpeer,
            device_id_type=pl.DeviceIdType.MESH,
        )
        pl.semaphore_wait(barrier, 2)

        zoff = my_x * half
        xoff = (1 - my_x) * half

        def cols(head):
            return pl.ds(head * d, d)

        z_rdmas = []
        for j in range(half):
            for t, (src, dst) in enumerate(((k_ref, kr), (v_ref, vr))):
                r = pltpu.make_async_remote_copy(
                    src_ref=src.at[:, cols(zoff + j)],
                    dst_ref=dst.at[:, cols(zoff + j)],
                    send_sem=zs.at[t, j],
                    recv_sem=zr.at[t, j],
                    device_id=zpeer,
                    device_id_type=pl.DeviceIdType.MESH,
                )
                r.start()
                z_rdmas.append(r)

        def compute_head(head):
            c = cols(head)
            q = q_ref[:, c]
            s1 = lax.dot_general(q, k_ref[:, c], NT,
                                 preferred_element_type=f32) * scale
            s2 = lax.dot_general(q, kr[:, c], NT,
                                 preferred_element_type=f32) * scale
            p1 = jnp.exp(s1)
            p2 = jnp.exp(s2)
            l = (jnp.sum(p1, axis=-1, keepdims=True)
                 + jnp.sum(p2, axis=-1, keepdims=True))
            o = (lax.dot_general(p1.astype(jnp.bfloat16), v_ref[:, c], NN,
                                 preferred_element_type=f32)
                 + lax.dot_general(p2.astype(jnp.bfloat16), vr[:, c], NN,
                                   preferred_element_type=f32))
            o_ref[:, c] = o / l

        x_rdmas = []
        for j in range(half):
            for t, (buf,) in enumerate(((kr,), (vr,))):
                z_rdmas[2 * j + t].wait_recv()
                r = pltpu.make_async_remote_copy(
                    src_ref=buf.at[:, cols(zoff + j)],
                    dst_ref=buf.at[:, cols(zoff + j)],
                    send_sem=xs.at[t, j],
                    recv_sem=xr.at[t, j],
                    device_id=xpeer,
                    device_id_type=pl.DeviceIdType.MESH,
                )
                r.start()
                x_rdmas.append(r)
            compute_head(zoff + j)

        for j in range(half):
            x_rdmas[2 * j].wait_recv()
            x_rdmas[2 * j + 1].wait_recv()
            compute_head(xoff + j)

        for r in z_rdmas + x_rdmas:
            r.wait_send()

    out = pl.pallas_call(
        body,
        out_shape=jax.ShapeDtypeStruct((s, hd), f32),
        in_specs=[pl.BlockSpec(memory_space=pltpu.MemorySpace.VMEM)] * 3,
        out_specs=pl.BlockSpec(memory_space=pltpu.MemorySpace.VMEM),
        scratch_shapes=[
            pltpu.VMEM((s, hd), jnp.bfloat16),
            pltpu.VMEM((s, hd), jnp.bfloat16),
            pltpu.SemaphoreType.DMA((2, half)),
            pltpu.SemaphoreType.DMA((2, half)),
            pltpu.SemaphoreType.DMA((2, half)),
            pltpu.SemaphoreType.DMA((2, half)),
        ],
        compiler_params=pltpu.CompilerParams(
            collective_id=0, has_side_effects=True
        ),
    )(
        Q.reshape(s, hd).astype(jnp.bfloat16),
        K.reshape(s, hd).astype(jnp.bfloat16),
        V.reshape(s, hd).astype(jnp.bfloat16),
    )
    return out.reshape(b, s, h, d)


# device time: 92314 ns/iter; 3.4139x vs baseline; 1.1195x over previous
import jax
import jax.numpy as jnp
from jax import lax
from jax.experimental import pallas as pl
from jax.experimental.pallas import tpu as pltpu


def kernel(Q, K, V):
    b, s, h, d = Q.shape
    hd = h * d
    half = h // 2
    rows = s // 2
    scale = d ** -0.5
    f32 = jnp.float32
    bf16 = jnp.bfloat16

    NT = (((1,), (1,)), ((), ()))
    NN = (((1,), (0,)), ((), ()))

    def body(q_ref, k_ref, v_ref, o_ref, kr, vr, ob_s, ob_r,
             zs, zr, xs, xr, ys, yr):
        my_x = lax.axis_index("x")
        my_y = lax.axis_index("y")
        my_z = lax.axis_index("z")
        zpeer = (my_x, my_y, 1 - my_z)
        xpeer = (1 - my_x, my_y, my_z)
        ypeer = (my_x, 1 - my_y, my_z)

        barrier = pltpu.get_barrier_semaphore()
        for peer in (zpeer, xpeer, ypeer):
            pl.semaphore_signal(
                barrier, inc=1, device_id=peer,
                device_id_type=pl.DeviceIdType.MESH,
            )
        pl.semaphore_wait(barrier, 3)

        zoff = my_x * half
        xoff = (1 - my_x) * half
        my_rows = pl.ds(my_y * rows, rows)
        peer_rows = pl.ds((1 - my_y) * rows, rows)

        def cols(head):
            return pl.ds(head * d, d)

        z_rdmas = []
        for j in range(half):
            for t, (src, dst) in enumerate(((k_ref, kr), (v_ref, vr))):
                r = pltpu.make_async_remote_copy(
                    src_ref=src.at[:, cols(zoff + j)],
                    dst_ref=dst.at[:, cols(zoff + j)],
                    send_sem=zs.at[t, j],
                    recv_sem=zr.at[t, j],
                    device_id=zpeer,
                    device_id_type=pl.DeviceIdType.MESH,
                )
                r.start()
                z_rdmas.append(r)

        y_rdmas = []

        def compute_head(head, sem_idx):
            c = cols(head)
            q = q_ref[my_rows, c]
            s1 = lax.dot_general(q, k_ref[:, c], NT,
                                 preferred_element_type=f32)
            s2 = lax.dot_general(q, kr[:, c], NT,
                                 preferred_element_type=f32)
            p1 = jnp.exp(s1)
            p2 = jnp.exp(s2)
            l = (jnp.sum(p1, axis=-1, keepdims=True)
                 + jnp.sum(p2, axis=-1, keepdims=True))
            o = (lax.dot_general(p1.astype(bf16), v_ref[:, c], NN,
                                 preferred_element_type=f32)
                 + lax.dot_general(p2.astype(bf16), vr[:, c], NN,
                                   preferred_element_type=f32)) / l
            o_ref[my_rows, c] = o
            ob_s[:, c] = o.astype(bf16)
            r = pltpu.make_async_remote_copy(
                src_ref=ob_s.at[:, c],
                dst_ref=ob_r.at[:, c],
                send_sem=ys.at[sem_idx],
                recv_sem=yr.at[sem_idx],
                device_id=ypeer,
                device_id_type=pl.DeviceIdType.MESH,
            )
            r.start()
            y_rdmas.append((r, head))

        x_rdmas = []
        for j in range(half):
            for t, buf in enumerate((kr, vr)):
                z_rdmas[2 * j + t].wait_recv()
                r = pltpu.make_async_remote_copy(
                    src_ref=buf.at[:, cols(zoff + j)],
                    dst_ref=buf.at[:, cols(zoff + j)],
                    send_sem=xs.at[t, j],
                    recv_sem=xr.at[t, j],
                    device_id=xpeer,
                    device_id_type=pl.DeviceIdType.MESH,
                )
                r.start()
                x_rdmas.append(r)
            compute_head(zoff + j, j)

        for j in range(half):
            x_rdmas[2 * j].wait_recv()
            x_rdmas[2 * j + 1].wait_recv()
            compute_head(xoff + j, half + j)

        for r, head in y_rdmas:
            r.wait_recv()
            c = cols(head)
            o_ref[peer_rows, c] = ob_r[:, c].astype(f32)

        for r in z_rdmas + x_rdmas:
            r.wait_send()
        for r, _ in y_rdmas:
            r.wait_send()

    out = pl.pallas_call(
        body,
        out_shape=jax.ShapeDtypeStruct((s, hd), f32),
        in_specs=[pl.BlockSpec(memory_space=pltpu.MemorySpace.VMEM)] * 3,
        out_specs=pl.BlockSpec(memory_space=pltpu.MemorySpace.VMEM),
        scratch_shapes=[
            pltpu.VMEM((s, hd), bf16),
            pltpu.VMEM((s, hd), bf16),
            pltpu.VMEM((rows, hd), bf16),
            pltpu.VMEM((rows, hd), bf16),
            pltpu.SemaphoreType.DMA((2, half)),
            pltpu.SemaphoreType.DMA((2, half)),
            pltpu.SemaphoreType.DMA((2, half)),
            pltpu.SemaphoreType.DMA((2, half)),
            pltpu.SemaphoreType.DMA((h,)),
            pltpu.SemaphoreType.DMA((h,)),
        ],
        compiler_params=pltpu.CompilerParams(
            collective_id=0, has_side_effects=True
        ),
    )(
        (Q.reshape(s, hd) * scale).astype(bf16),
        K.reshape(s, hd).astype(bf16),
        V.reshape(s, hd).astype(bf16),
    )
    return out.reshape(b, s, h, d)


# device time: 85875 ns/iter; 3.6699x vs baseline; 1.0750x over previous
import jax
import jax.numpy as jnp
from jax import lax
from jax.experimental import pallas as pl
from jax.experimental.pallas import tpu as pltpu


def kernel(Q, K, V):
    b, s, h, d = Q.shape
    hd = h * d
    half = h // 2
    rows = s // 2
    scale = d ** -0.5
    f32 = jnp.float32
    bf16 = jnp.bfloat16

    NT = (((1,), (1,)), ((), ()))
    NN = (((1,), (0,)), ((), ()))

    def body(q_ref, k_ref, v_ref, o_ref, kr, vr, zs, zr, xs, xr, ys, yr):
        my_x = lax.axis_index("x")
        my_y = lax.axis_index("y")
        my_z = lax.axis_index("z")
        zpeer = (my_x, my_y, 1 - my_z)
        xpeer = (1 - my_x, my_y, my_z)
        ypeer = (my_x, 1 - my_y, my_z)

        barrier = pltpu.get_barrier_semaphore()
        for peer in (zpeer, xpeer, ypeer):
            pl.semaphore_signal(
                barrier, inc=1, device_id=peer,
                device_id_type=pl.DeviceIdType.MESH,
            )
        pl.semaphore_wait(barrier, 3)

        zoff = my_x * half
        xoff = (1 - my_x) * half
        my_rows = pl.ds(my_y * rows, rows)

        def cols(head):
            return pl.ds(head * d, d)

        z_rdmas = []
        for j in range(half):
            for t, (src, dst) in enumerate(((k_ref, kr), (v_ref, vr))):
                r = pltpu.make_async_remote_copy(
                    src_ref=src.at[:, cols(zoff + j)],
                    dst_ref=dst.at[:, cols(zoff + j)],
                    send_sem=zs.at[t, j],
                    recv_sem=zr.at[t, j],
                    device_id=zpeer,
                    device_id_type=pl.DeviceIdType.MESH,
                )
                r.start()
                z_rdmas.append(r)

        y_rdmas = []

        def compute_head(head, sem_idx):
            c = cols(head)
            q = q_ref[my_rows, c]
            s1 = lax.dot_general(q, k_ref[:, c], NT,
                                 preferred_element_type=f32)
            s2 = lax.dot_general(q, kr[:, c], NT,
                                 preferred_element_type=f32)
            p1 = jnp.exp(s1)
            p2 = jnp.exp(s2)
            l = (jnp.sum(p1, axis=-1, keepdims=True)
                 + jnp.sum(p2, axis=-1, keepdims=True))
            o = (lax.dot_general(p1.astype(bf16), v_ref[:, c], NN,
                                 preferred_element_type=f32)
                 + lax.dot_general(p2.astype(bf16), vr[:, c], NN,
                                   preferred_element_type=f32)) / l
            o_ref[my_rows, c] = o
            r = pltpu.make_async_remote_copy(
                src_ref=o_ref.at[my_rows, c],
                dst_ref=o_ref.at[my_rows, c],
                send_sem=ys.at[sem_idx],
                recv_sem=yr.at[sem_idx],
                device_id=ypeer,
                device_id_type=pl.DeviceIdType.MESH,
            )
            r.start()
            y_rdmas.append(r)

        def do_x_head(j):
            x_rdmas[2 * j].wait_recv()
            x_rdmas[2 * j + 1].wait_recv()
            compute_head(xoff + j, half + j)

        x_rdmas = []
        for j in range(half):
            for t, buf in enumerate((kr, vr)):
                z_rdmas[2 * j + t].wait_recv()
                r = pltpu.make_async_remote_copy(
                    src_ref=buf.at[:, cols(zoff + j)],
                    dst_ref=buf.at[:, cols(zoff + j)],
                    send_sem=xs.at[t, j],
                    recv_sem=xr.at[t, j],
                    device_id=xpeer,
                    device_id_type=pl.DeviceIdType.MESH,
                )
                r.start()
                x_rdmas.append(r)
            compute_head(zoff + j, j)
            if j >= 2:
                do_x_head(j - 2)
        for j in range(half - 2, half):
            do_x_head(j)

        for r in y_rdmas:
            r.wait_recv()

        for r in z_rdmas + x_rdmas + y_rdmas:
            r.wait_send()

    out = pl.pallas_call(
        body,
        out_shape=jax.ShapeDtypeStruct((s, hd), f32),
        in_specs=[pl.BlockSpec(memory_space=pltpu.MemorySpace.VMEM)] * 3,
        out_specs=pl.BlockSpec(memory_space=pltpu.MemorySpace.VMEM),
        scratch_shapes=[
            pltpu.VMEM((s, hd), bf16),
            pltpu.VMEM((s, hd), bf16),
            pltpu.SemaphoreType.DMA((2, half)),
            pltpu.SemaphoreType.DMA((2, half)),
            pltpu.SemaphoreType.DMA((2, half)),
            pltpu.SemaphoreType.DMA((2, half)),
            pltpu.SemaphoreType.DMA((h,)),
            pltpu.SemaphoreType.DMA((h,)),
        ],
        compiler_params=pltpu.CompilerParams(
            collective_id=0, has_side_effects=True
        ),
    )(
        (Q.reshape(s, hd) * scale).astype(bf16),
        K.reshape(s, hd).astype(bf16),
        V.reshape(s, hd).astype(bf16),
    )
    return out.reshape(b, s, h, d)


# device time: 81746 ns/iter; 3.8553x vs baseline; 1.0505x over previous
import jax
import jax.numpy as jnp
from jax import lax
from jax.experimental import pallas as pl
from jax.experimental.pallas import tpu as pltpu


def kernel(Q, K, V):
    b, s, h, d = Q.shape
    hd = h * d
    qr_rows = s // 4
    scale = d ** -0.5
    f32 = jnp.float32
    bf16 = jnp.bfloat16
    LAG = 4

    NT = (((1,), (1,)), ((), ()))
    NN = (((1,), (0,)), ((), ()))

    def body(q_ref, k_ref, v_ref, o_ref, qr, obz, oq, lz, lq, lbuf,
             qz_s, qz_r, oz_s, oz_r, xa_s, xa_r, yb_s, yb_r):
        my_x = lax.axis_index("x")
        my_y = lax.axis_index("y")
        my_z = lax.axis_index("z")
        zpeer = (my_x, my_y, 1 - my_z)
        xpeer = (1 - my_x, my_y, my_z)
        ypeer = (my_x, 1 - my_y, my_z)

        barrier = pltpu.get_barrier_semaphore()
        for peer in (zpeer, xpeer, ypeer):
            pl.semaphore_signal(
                barrier, inc=1, device_id=peer,
                device_id_type=pl.DeviceIdType.MESH,
            )
        pl.semaphore_wait(barrier, 3)

        p_me = 2 * my_x + my_y
        p_x = 2 * (1 - my_x) + my_y
        p_y = 2 * my_x + (1 - my_y)
        p_d = 2 * (1 - my_x) + (1 - my_y)
        slot_rows = [p_me, p_x, p_y, p_d]

        def cols(head):
            return pl.ds(head * d, d)

        def rows(pidx):
            return pl.ds(pidx * qr_rows, qr_rows)

        def rdma(src, dst, ssem, rsem, peer):
            r = pltpu.make_async_remote_copy(
                src_ref=src, dst_ref=dst, send_sem=ssem, recv_sem=rsem,
                device_id=peer, device_id_type=pl.DeviceIdType.MESH,
            )
            r.start()
            return r

        started = []

        qz = []
        for j in range(h):
            qz.append(rdma(q_ref.at[rows(p_me), cols(j)],
                           qr.at[:, cols(j)],
                           qz_s.at[j], qz_r.at[j], zpeer))
        started += qz

        oz, a_fw, bmy_fw = [], [], []

        def fwd_a(i):
            oz[i].wait_recv()
            a_fw.append(rdma(oq.at[0, :, cols(i)], oq.at[1, :, cols(i)],
                             xa_s.at[i], xa_r.at[i], xpeer))
            bmy_fw.append(rdma(oq.at[0, :, cols(i)], oq.at[2, :, cols(i)],
                               yb_s.at[i], yb_r.at[i], ypeer))

        for j in range(h):
            c = cols(j)
            qz[j].wait_recv()
            q2 = qr[:, c]
            s2 = lax.dot_general(q2, k_ref[:, c], NT,
                                 preferred_element_type=f32)
            p2 = jnp.exp(s2)
            lz[:, j:j + 1] = jnp.sum(p2, axis=-1, keepdims=True)
            o2 = lax.dot_general(p2.astype(bf16), v_ref[:, c], NN,
                                 preferred_element_type=f32)
            obz[:, c] = o2.astype(bf16)
            oz.append(rdma(obz.at[:, c], oq.at[0, :, c],
                           oz_s.at[j], oz_r.at[j], zpeer))
            if j >= LAG:
                fwd_a(j - LAG)
        oz_l = rdma(lz, lq.at[0], oz_s.at[h], oz_r.at[h], zpeer)
        started.append(oz_l)
        for i in range(h - LAG, h):
            fwd_a(i)
        oz_l.wait_recv()
        a_fw.append(rdma(lq.at[0], lq.at[1], xa_s.at[h], xa_r.at[h], xpeer))
        bmy_fw.append(rdma(lq.at[0], lq.at[2], yb_s.at[2 * h],
                           yb_r.at[2 * h], ypeer))
        started += oz + a_fw + bmy_fw

        bx_fw = []
        for j in range(h):
            c = cols(j)
            q1 = q_ref[:, c]
            s1 = lax.dot_general(q1, k_ref[:, c], NT,
                                 preferred_element_type=f32)
            p1 = jnp.exp(s1)
            lbuf[:, j:j + 1] = jnp.sum(p1, axis=-1, keepdims=True)
            o_ref[:, c] = lax.dot_general(p1.astype(bf16), v_ref[:, c], NN,
                                          preferred_element_type=f32)
            a_fw[j].wait_recv()
            bx_fw.append(rdma(oq.at[1, :, c], oq.at[3, :, c],
                              yb_s.at[h + j], yb_r.at[h + j], ypeer))
        a_fw[h].wait_recv()
        bx_fw.append(rdma(lq.at[1], lq.at[3], yb_s.at[2 * h + 1],
                          yb_r.at[2 * h + 1], ypeer))
        started += bx_fw
        bmy_fw[h].wait_recv()
        bx_fw[h].wait_recv()

        for j in range(h):
            c = cols(j)
            bmy_fw[j].wait_recv()
            bx_fw[j].wait_recv()
            for slot in range(4):
                rsl = rows(slot_rows[slot])
                o1 = o_ref[rsl, c]
                l1 = lbuf[rsl, j:j + 1]
                o2 = oq[slot, :, c].astype(f32)
                l2 = lq[slot, :, j:j + 1]
                o_ref[rsl, c] = (o1 + o2) / (l1 + l2)

        for r in started:
            r.wait_send()

    out = pl.pallas_call(
        body,
        out_shape=jax.ShapeDtypeStruct((s, hd), f32),
        in_specs=[pl.BlockSpec(memory_space=pltpu.MemorySpace.VMEM)] * 3,
        out_specs=pl.BlockSpec(memory_space=pltpu.MemorySpace.VMEM),
        scratch_shapes=[
            pltpu.VMEM((qr_rows, hd), bf16),
            pltpu.VMEM((qr_rows, hd), bf16),
            pltpu.VMEM((4, qr_rows, hd), bf16),
            pltpu.VMEM((qr_rows, h), f32),
            pltpu.VMEM((4, qr_rows, h), f32),
            pltpu.VMEM((s, h), f32),
            pltpu.SemaphoreType.DMA((h,)),
            pltpu.SemaphoreType.DMA((h,)),
            pltpu.SemaphoreType.DMA((h + 1,)),
            pltpu.SemaphoreType.DMA((h + 1,)),
            pltpu.SemaphoreType.DMA((h + 1,)),
            pltpu.SemaphoreType.DMA((h + 1,)),
            pltpu.SemaphoreType.DMA((2 * h + 2,)),
            pltpu.SemaphoreType.DMA((2 * h + 2,)),
        ],
        compiler_params=pltpu.CompilerParams(
            collective_id=0, has_side_effects=True
        ),
    )(
        (Q.reshape(s, hd) * scale).astype(bf16),
        K.reshape(s, hd).astype(bf16),
        V.reshape(s, hd).astype(bf16),
    )
    return out.reshape(b, s, h, d)


# device time: 74314 ns/iter; 4.2408x vs baseline; 1.1000x over previous
import jax
import jax.numpy as jnp
from jax import lax
from jax.experimental import pallas as pl
from jax.experimental.pallas import tpu as pltpu


def kernel(Q, K, V):
    b, s, h, d = Q.shape
    hd = h * d
    qr_rows = s // 4
    scale = d ** -0.5
    f32 = jnp.float32
    bf16 = jnp.bfloat16
    LAG = 4

    NT = (((1,), (1,)), ((), ()))
    NN = (((1,), (0,)), ((), ()))

    def body(q_ref, k_ref, v_ref, o_ref, qr, obz, oq, lz, lq, lbuf,
             qz_s, qz_r, oz_s, oz_r, xa_s, xa_r, yb_s, yb_r):
        my_x = lax.axis_index("x")
        my_y = lax.axis_index("y")
        my_z = lax.axis_index("z")
        zpeer = (my_x, my_y, 1 - my_z)
        xpeer = (1 - my_x, my_y, my_z)
        ypeer = (my_x, 1 - my_y, my_z)

        barrier = pltpu.get_barrier_semaphore()
        for peer in (zpeer, xpeer, ypeer):
            pl.semaphore_signal(
                barrier, inc=1, device_id=peer,
                device_id_type=pl.DeviceIdType.MESH,
            )
        pl.semaphore_wait(barrier, 3)

        p_me = 2 * my_x + my_y
        p_x = 2 * (1 - my_x) + my_y
        p_y = 2 * my_x + (1 - my_y)
        p_d = 2 * (1 - my_x) + (1 - my_y)
        slot_rows = [p_me, p_x, p_y, p_d]

        def cols(head):
            return pl.ds(head * d, d)

        def rows(pidx):
            return pl.ds(pidx * qr_rows, qr_rows)

        def rdma(src, dst, ssem, rsem, peer):
            r = pltpu.make_async_remote_copy(
                src_ref=src, dst_ref=dst, send_sem=ssem, recv_sem=rsem,
                device_id=peer, device_id_type=pl.DeviceIdType.MESH,
            )
            r.start()
            return r

        started = []

        qz = []
        for j in range(h):
            qz.append(rdma(q_ref.at[rows(p_me), cols(j)],
                           qr.at[:, cols(j)],
                           qz_s.at[j], qz_r.at[j], zpeer))
        started += qz

        oz, a_fw, bmy_fw = [], [], []

        def fwd_a(i):
            oz[i].wait_recv()
            a_fw.append(rdma(oq.at[0, :, cols(i)], oq.at[1, :, cols(i)],
                             xa_s.at[i], xa_r.at[i], xpeer))
            bmy_fw.append(rdma(oq.at[0, :, cols(i)], oq.at[2, :, cols(i)],
                               yb_s.at[i], yb_r.at[i], ypeer))

        for j in range(h):
            c = cols(j)
            qz[j].wait_recv()
            q2 = qr[:, c]
            s2 = lax.dot_general(q2, k_ref[:, c], NT,
                                 preferred_element_type=f32)
            p2 = jnp.exp(s2)
            lz[:, j:j + 1] = jnp.sum(p2, axis=-1, keepdims=True)
            o2 = lax.dot_general(p2.astype(bf16), v_ref[:, c], NN,
                                 preferred_element_type=f32)
            obz[:, c] = o2.astype(bf16)
            oz.append(rdma(obz.at[:, c], oq.at[0, :, c],
                           oz_s.at[j], oz_r.at[j], zpeer))
        oz_l = rdma(lz, lq.at[0], oz_s.at[h], oz_r.at[h], zpeer)
        started.append(oz_l)

        bx_fw = []

        def fwd_bx(i):
            a_fw[i].wait_recv()
            bx_fw.append(rdma(oq.at[1, :, cols(i)], oq.at[3, :, cols(i)],
                              yb_s.at[h + i], yb_r.at[h + i], ypeer))

        for j in range(h):
            c = cols(j)
            fwd_a(j)
            q1 = q_ref[:, c]
            s1 = lax.dot_general(q1, k_ref[:, c], NT,
                                 preferred_element_type=f32)
            p1 = jnp.exp(s1)
            lbuf[:, j:j + 1] = jnp.sum(p1, axis=-1, keepdims=True)
            o_ref[:, c] = lax.dot_general(p1.astype(bf16), v_ref[:, c], NN,
                                          preferred_element_type=f32)
            if j >= LAG:
                fwd_bx(j - LAG)
        for i in range(h - LAG, h):
            fwd_bx(i)

        oz_l.wait_recv()
        a_fw.append(rdma(lq.at[0], lq.at[1], xa_s.at[h], xa_r.at[h], xpeer))
        bmy_fw.append(rdma(lq.at[0], lq.at[2], yb_s.at[2 * h],
                           yb_r.at[2 * h], ypeer))
        a_fw[h].wait_recv()
        bx_fw.append(rdma(lq.at[1], lq.at[3], yb_s.at[2 * h + 1],
                          yb_r.at[2 * h + 1], ypeer))
        started += oz + a_fw + bmy_fw + bx_fw
        bmy_fw[h].wait_recv()
        bx_fw[h].wait_recv()

        for j in range(h):
            c = cols(j)
            bmy_fw[j].wait_recv()
            bx_fw[j].wait_recv()
            for slot in range(4):
                rsl = rows(slot_rows[slot])
                o1 = o_ref[rsl, c]
                l1 = lbuf[rsl, j:j + 1]
                o2 = oq[slot, :, c].astype(f32)
                l2 = lq[slot, :, j:j + 1]
                o_ref[rsl, c] = (o1 + o2) / (l1 + l2)

        for r in started:
            r.wait_send()

    out = pl.pallas_call(
        body,
        out_shape=jax.ShapeDtypeStruct((s, hd), f32),
        in_specs=[pl.BlockSpec(memory_space=pltpu.MemorySpace.VMEM)] * 3,
        out_specs=pl.BlockSpec(memory_space=pltpu.MemorySpace.VMEM),
        scratch_shapes=[
            pltpu.VMEM((qr_rows, hd), bf16),
            pltpu.VMEM((qr_rows, hd), bf16),
            pltpu.VMEM((4, qr_rows, hd), bf16),
            pltpu.VMEM((qr_rows, h), f32),
            pltpu.VMEM((4, qr_rows, h), f32),
            pltpu.VMEM((s, h), f32),
            pltpu.SemaphoreType.DMA((h,)),
            pltpu.SemaphoreType.DMA((h,)),
            pltpu.SemaphoreType.DMA((h + 1,)),
            pltpu.SemaphoreType.DMA((h + 1,)),
            pltpu.SemaphoreType.DMA((h + 1,)),
            pltpu.SemaphoreType.DMA((h + 1,)),
            pltpu.SemaphoreType.DMA((2 * h + 2,)),
            pltpu.SemaphoreType.DMA((2 * h + 2,)),
        ],
        compiler_params=pltpu.CompilerParams(
            collective_id=0, has_side_effects=True
        ),
    )(
        (Q.reshape(s, hd) * scale).astype(bf16),
        K.reshape(s, hd).astype(bf16),
        V.reshape(s, hd).astype(bf16),
    )
    return out.reshape(b, s, h, d)


# device time: 72888 ns/iter; 4.3238x vs baseline; 1.0196x over previous
import jax
import jax.numpy as jnp
from jax import lax
from jax.experimental import pallas as pl
from jax.experimental.pallas import tpu as pltpu


def kernel(Q, K, V):
    b, s, h, d = Q.shape
    hd = h * d
    qr_rows = s // 4
    scale = d ** -0.5
    f32 = jnp.float32
    bf16 = jnp.bfloat16
    HPC = 2
    nch = h // HPC
    LAG = 2

    NT = (((1,), (1,)), ((), ()))
    NN = (((1,), (0,)), ((), ()))

    def body(q_ref, k_ref, v_ref, o_ref, qr, obz, oq, lz, lq, lbuf,
             qz_s, qz_r, oz_s, oz_r, xa_s, xa_r, yb_s, yb_r):
        my_x = lax.axis_index("x")
        my_y = lax.axis_index("y")
        my_z = lax.axis_index("z")
        zpeer = (my_x, my_y, 1 - my_z)
        xpeer = (1 - my_x, my_y, my_z)
        ypeer = (my_x, 1 - my_y, my_z)

        barrier = pltpu.get_barrier_semaphore()
        for peer in (zpeer, xpeer, ypeer):
            pl.semaphore_signal(
                barrier, inc=1, device_id=peer,
                device_id_type=pl.DeviceIdType.MESH,
            )
        pl.semaphore_wait(barrier, 3)

        p_me = 2 * my_x + my_y
        p_x = 2 * (1 - my_x) + my_y
        p_y = 2 * my_x + (1 - my_y)
        p_d = 2 * (1 - my_x) + (1 - my_y)
        slot_rows = [p_me, p_x, p_y, p_d]

        def cols(head):
            return pl.ds(head * d, d)

        def ccols(chunk):
            return pl.ds(chunk * HPC * d, HPC * d)

        def rows(pidx):
            return pl.ds(pidx * qr_rows, qr_rows)

        def rdma(src, dst, ssem, rsem, peer):
            r = pltpu.make_async_remote_copy(
                src_ref=src, dst_ref=dst, send_sem=ssem, recv_sem=rsem,
                device_id=peer, device_id_type=pl.DeviceIdType.MESH,
            )
            r.start()
            return r

        started = []

        qz = []
        for i in range(nch):
            qz.append(rdma(q_ref.at[rows(p_me), ccols(i)],
                           qr.at[:, ccols(i)],
                           qz_s.at[i], qz_r.at[i], zpeer))
        started += qz

        oz, a_fw, bmy_fw = [], [], []

        def fwd_a(i):
            oz[i].wait_recv()
            a_fw.append(rdma(oq.at[0, :, ccols(i)], oq.at[1, :, ccols(i)],
                             xa_s.at[i], xa_r.at[i], xpeer))
            bmy_fw.append(rdma(oq.at[0, :, ccols(i)], oq.at[2, :, ccols(i)],
                               yb_s.at[i], yb_r.at[i], ypeer))

        for i in range(nch):
            qz[i].wait_recv()
            for j in range(HPC * i, HPC * (i + 1)):
                c = cols(j)
                q2 = qr[:, c]
                s2 = lax.dot_general(q2, k_ref[:, c], NT,
                                     preferred_element_type=f32)
                p2 = jnp.exp(s2)
                lz[:, j:j + 1] = jnp.sum(p2, axis=-1, keepdims=True)
                o2 = lax.dot_general(p2.astype(bf16), v_ref[:, c], NN,
                                     preferred_element_type=f32)
                obz[:, c] = o2.astype(bf16)
            oz.append(rdma(obz.at[:, ccols(i)], oq.at[0, :, ccols(i)],
                           oz_s.at[i], oz_r.at[i], zpeer))
        oz_l = rdma(lz, lq.at[0], oz_s.at[nch], oz_r.at[nch], zpeer)
        started.append(oz_l)

        bx_fw = []

        def fwd_bx(i):
            a_fw[i].wait_recv()
            bx_fw.append(rdma(oq.at[1, :, ccols(i)], oq.at[3, :, ccols(i)],
                              yb_s.at[nch + i], yb_r.at[nch + i], ypeer))

        for i in range(nch):
            fwd_a(i)
            for j in range(HPC * i, HPC * (i + 1)):
                c = cols(j)
                q1 = q_ref[:, c]
                s1 = lax.dot_general(q1, k_ref[:, c], NT,
                                     preferred_element_type=f32)
                p1 = jnp.exp(s1)
                lbuf[:, j:j + 1] = jnp.sum(p1, axis=-1, keepdims=True)
                o_ref[:, c] = lax.dot_general(
                    p1.astype(bf16), v_ref[:, c], NN,
                    preferred_element_type=f32)
            if i >= LAG:
                fwd_bx(i - LAG)
        for i in range(nch - LAG, nch):
            fwd_bx(i)

        oz_l.wait_recv()
        a_fw.append(rdma(lq.at[0], lq.at[1], xa_s.at[nch], xa_r.at[nch],
                         xpeer))
        bmy_fw.append(rdma(lq.at[0], lq.at[2], yb_s.at[2 * nch],
                           yb_r.at[2 * nch], ypeer))
        a_fw[nch].wait_recv()
        bx_fw.append(rdma(lq.at[1], lq.at[3], yb_s.at[2 * nch + 1],
                          yb_r.at[2 * nch + 1], ypeer))
        started += oz + a_fw + bmy_fw + bx_fw
        bmy_fw[nch].wait_recv()
        bx_fw[nch].wait_recv()

        for i in range(nch):
            bmy_fw[i].wait_recv()
            bx_fw[i].wait_recv()
            for j in range(HPC * i, HPC * (i + 1)):
                c = cols(j)
                for slot in range(4):
                    rsl = rows(slot_rows[slot])
                    o1 = o_ref[rsl, c]
                    l1 = lbuf[rsl, j:j + 1]
                    o2 = oq[slot, :, c].astype(f32)
                    l2 = lq[slot, :, j:j + 1]
                    o_ref[rsl, c] = (o1 + o2) / (l1 + l2)

        for r in started:
            r.wait_send()

    out = pl.pallas_call(
        body,
        out_shape=jax.ShapeDtypeStruct((s, hd), f32),
        in_specs=[pl.BlockSpec(memory_space=pltpu.MemorySpace.VMEM)] * 3,
        out_specs=pl.BlockSpec(memory_space=pltpu.MemorySpace.VMEM),
        scratch_shapes=[
            pltpu.VMEM((qr_rows, hd), bf16),
            pltpu.VMEM((qr_rows, hd), bf16),
            pltpu.VMEM((4, qr_rows, hd), bf16),
            pltpu.VMEM((qr_rows, h), f32),
            pltpu.VMEM((4, qr_rows, h), f32),
            pltpu.VMEM((s, h), f32),
            pltpu.SemaphoreType.DMA((nch,)),
            pltpu.SemaphoreType.DMA((nch,)),
            pltpu.SemaphoreType.DMA((nch + 1,)),
            pltpu.SemaphoreType.DMA((nch + 1,)),
            pltpu.SemaphoreType.DMA((nch + 1,)),
            pltpu.SemaphoreType.DMA((nch + 1,)),
            pltpu.SemaphoreType.DMA((2 * nch + 2,)),
            pltpu.SemaphoreType.DMA((2 * nch + 2,)),
        ],
        compiler_params=pltpu.CompilerParams(
            collective_id=0, has_side_effects=True
        ),
    )(
        (Q.reshape(s, hd) * scale).astype(bf16),
        K.reshape(s, hd).astype(bf16),
        V.reshape(s, hd).astype(bf16),
    )
    return out.reshape(b, s, h, d)


# device time: 68148 ns/iter; 4.6245x vs baseline; 1.0696x over previous
import jax
import jax.numpy as jnp
from jax import lax
from jax.experimental import pallas as pl
from jax.experimental.pallas import tpu as pltpu


def kernel(Q, K, V):
    b, s, h, d = Q.shape
    hd = h * d
    qr_rows = s // 4
    scale = d ** -0.5
    f32 = jnp.float32
    bf16 = jnp.bfloat16
    HPC = 2
    nch = h // HPC

    NT = (((1,), (1,)), ((), ()))
    NN = (((1,), (0,)), ((), ()))

    def body(q_ref, k_ref, v_ref, o_ref, qr, obz, oq0, ob, lz, lq0,
             qz_s, qz_r, oz_s, oz_r, xg_s, xg_r, yg_s, yg_r):
        my_x = lax.axis_index("x")
        my_y = lax.axis_index("y")
        my_z = lax.axis_index("z")
        zpeer = (my_x, my_y, 1 - my_z)
        xpeer = (1 - my_x, my_y, my_z)
        ypeer = (my_x, 1 - my_y, my_z)

        barrier = pltpu.get_barrier_semaphore()
        for peer in (zpeer, xpeer, ypeer):
            pl.semaphore_signal(
                barrier, inc=1, device_id=peer,
                device_id_type=pl.DeviceIdType.MESH,
            )
        pl.semaphore_wait(barrier, 3)

        p_me = 2 * my_x + my_y
        p_x = 2 * (1 - my_x) + my_y
        p_y = 2 * my_x + (1 - my_y)
        p_d = 2 * (1 - my_x) + (1 - my_y)

        def cols(head):
            return pl.ds(head * d, d)

        def ccols(chunk):
            return pl.ds(chunk * HPC * d, HPC * d)

        def rows(pidx):
            return pl.ds(pidx * qr_rows, qr_rows)

        def rdma(src, dst, ssem, rsem, peer):
            r = pltpu.make_async_remote_copy(
                src_ref=src, dst_ref=dst, send_sem=ssem, recv_sem=rsem,
                device_id=peer, device_id_type=pl.DeviceIdType.MESH,
            )
            r.start()
            return r

        started = []

        qz = []
        for i in range(nch):
            qz.append(rdma(q_ref.at[rows(p_me), ccols(i)],
                           qr.at[:, ccols(i)],
                           qz_s.at[i], qz_r.at[i], zpeer))
        started += qz

        oz = []
        for i in range(nch):
            qz[i].wait_recv()
            for j in range(HPC * i, HPC * (i + 1)):
                c = cols(j)
                q2 = qr[:, c]
                s2 = lax.dot_general(q2, k_ref[:, c], NT,
                                     preferred_element_type=f32)
                p2 = jnp.exp(s2)
                lz[:, j:j + 1] = jnp.sum(p2, axis=-1, keepdims=True)
                o2 = lax.dot_general(p2.astype(bf16), v_ref[:, c], NN,
                                     preferred_element_type=f32)
                obz[:, c] = o2.astype(bf16)
            oz.append(rdma(obz.at[:, ccols(i)], oq0.at[:, ccols(i)],
                           oz_s.at[i], oz_r.at[i], zpeer))
        oz_l = rdma(lz, lq0, oz_s.at[nch], oz_r.at[nch], zpeer)
        started += oz
        started.append(oz_l)
        oz_l.wait_recv()

        xg, yg = [], []
        mrows = rows(p_me)
        for i in range(nch):
            oz[i].wait_recv()
            for j in range(HPC * i, HPC * (i + 1)):
                c = cols(j)
                q1 = q_ref[mrows, c]
                s1 = lax.dot_general(q1, k_ref[:, c], NT,
                                     preferred_element_type=f32)
                p1 = jnp.exp(s1)
                l1 = jnp.sum(p1, axis=-1, keepdims=True)
                o1 = lax.dot_general(p1.astype(bf16), v_ref[:, c], NN,
                                     preferred_element_type=f32)
                o = (o1 + oq0[:, c].astype(f32)) / (l1 + lq0[:, j:j + 1])
                o_ref[mrows, c] = o
                ob[mrows, c] = o.astype(bf16)
            xg.append(rdma(ob.at[mrows, ccols(i)], ob.at[mrows, ccols(i)],
                           xg_s.at[i], xg_r.at[i], xpeer))
            yg.append(rdma(ob.at[mrows, ccols(i)], ob.at[mrows, ccols(i)],
                           yg_s.at[i], yg_r.at[i], ypeer))
        started += xg + yg

        xd, yd = [], []
        for i in range(nch):
            c2 = ccols(i)
            xg[i].wait_recv()
            if i % 2 == 1:
                yd.append(rdma(ob.at[rows(p_x), c2], ob.at[rows(p_x), c2],
                               yg_s.at[nch + i // 2], yg_r.at[nch + i // 2],
                               ypeer))
            o_ref[rows(p_x), c2] = ob[rows(p_x), c2].astype(f32)
            yg[i].wait_recv()
            if i % 2 == 0:
                xd.append(rdma(ob.at[rows(p_y), c2], ob.at[rows(p_y), c2],
                               xg_s.at[nch + i // 2], xg_r.at[nch + i // 2],
                               xpeer))
            o_ref[rows(p_y), c2] = ob[rows(p_y), c2].astype(f32)
        started += xd + yd

        for i in range(nch):
            c2 = ccols(i)
            if i % 2 == 0:
                xd[i // 2].wait_recv()
            else:
                yd[i // 2].wait_recv()
            o_ref[rows(p_d), c2] = ob[rows(p_d), c2].astype(f32)

        for r in started:
            r.wait_send()

    out = pl.pallas_call(
        body,
        out_shape=jax.ShapeDtypeStruct((s, hd), f32),
        in_specs=[pl.BlockSpec(memory_space=pltpu.MemorySpace.VMEM)] * 3,
        out_specs=pl.BlockSpec(memory_space=pltpu.MemorySpace.VMEM),
        scratch_shapes=[
            pltpu.VMEM((qr_rows, hd), bf16),
            pltpu.VMEM((qr_rows, hd), bf16),
            pltpu.VMEM((qr_rows, hd), bf16),
            pltpu.VMEM((s, hd), bf16),
            pltpu.VMEM((qr_rows, h), f32),
            pltpu.VMEM((qr_rows, h), f32),
            pltpu.SemaphoreType.DMA((nch,)),
            pltpu.SemaphoreType.DMA((nch,)),
            pltpu.SemaphoreType.DMA((nch + 1,)),
            pltpu.SemaphoreType.DMA((nch + 1,)),
            pltpu.SemaphoreType.DMA((nch + nch // 2,)),
            pltpu.SemaphoreType.DMA((nch + nch // 2,)),
            pltpu.SemaphoreType.DMA((nch + nch // 2,)),
            pltpu.SemaphoreType.DMA((nch + nch // 2,)),
        ],
        compiler_params=pltpu.CompilerParams(
            collective_id=0, has_side_effects=True
        ),
    )(
        (Q.reshape(s, hd) * scale).astype(bf16),
        K.reshape(s, hd).astype(bf16),
        V.reshape(s, hd).astype(bf16),
    )
    return out.reshape(b, s, h, d)
